# baseline (speedup 1.0000x reference)
import functools

import jax
import jax.numpy as jnp
from jax import lax
from jax.experimental import pallas as pl
from jax.experimental.pallas import tpu as pltpu
from jax.experimental.pallas import tpu_sc as plsc

BM = 256
K = 128
KA = 64
NSUB = 16
NCORE = 2


def _tc1_body(x_ref, w_ref, h0, h1, h2, h3, auxs, auxd, maxs, maxd):
    xb = x_ref[...]
    w = w_ref[...]
    for hh, href in enumerate((h0, h1, h2, h3)):
        href[...] = jnp.dot(xb, w[:, hh * 128:(hh + 1) * 128],
                            preferred_element_type=jnp.float32)
    a_s = jnp.dot(xb, w[:, 512:640], preferred_element_type=jnp.float32)
    a_d = jnp.dot(xb, w[:, 640:768], preferred_element_type=jnp.float32)
    auxs[...] = a_s
    auxd[...] = a_d
    ms = jnp.max(a_s, axis=0, keepdims=True)
    md = jnp.max(a_d, axis=0, keepdims=True)
    i = pl.program_id(0)

    @pl.when(i == 0)
    def _():
        maxs[...] = ms
        maxd[...] = md

    @pl.when(i > 0)
    def _():
        maxs[...] = jnp.maximum(maxs[...], ms)
        maxd[...] = jnp.maximum(maxd[...], md)


def _tc1(xpad, wall, mpad):
    kp = xpad.shape[1]
    return pl.pallas_call(
        _tc1_body,
        grid=(mpad // BM,),
        in_specs=[
            pl.BlockSpec((BM, kp), lambda i: (i, 0)),
            pl.BlockSpec((kp, 768), lambda i: (0, 0)),
        ],
        out_specs=[
            pl.BlockSpec((BM, 128), lambda i: (i, 0)),
            pl.BlockSpec((BM, 128), lambda i: (i, 0)),
            pl.BlockSpec((BM, 128), lambda i: (i, 0)),
            pl.BlockSpec((BM, 128), lambda i: (i, 0)),
            pl.BlockSpec((BM, 128), lambda i: (i, 0)),
            pl.BlockSpec((BM, 128), lambda i: (i, 0)),
            pl.BlockSpec((1, 128), lambda i: (0, 0)),
            pl.BlockSpec((1, 128), lambda i: (0, 0)),
        ],
        out_shape=[
            jax.ShapeDtypeStruct((mpad, 128), jnp.float32),
            jax.ShapeDtypeStruct((mpad, 128), jnp.float32),
            jax.ShapeDtypeStruct((mpad, 128), jnp.float32),
            jax.ShapeDtypeStruct((mpad, 128), jnp.float32),
            jax.ShapeDtypeStruct((mpad, 128), jnp.float32),
            jax.ShapeDtypeStruct((mpad, 128), jnp.float32),
            jax.ShapeDtypeStruct((1, 128), jnp.float32),
            jax.ShapeDtypeStruct((1, 128), jnp.float32),
        ],
    )(xpad, wall)


def _make_sca(ep, ep_pad, mpad, nacc, heads):
    cpw = ep_pad // (KA * NCORE * NSUB)
    rows_per_sub = nacc // NSUB
    zslabs = [K] * (rows_per_sub // K) + ([rows_per_sub % K] if rows_per_sub % K else [])
    mesh = plsc.VectorSubcoreMesh(core_axis_name="c", subcore_axis_name="s")

    @functools.partial(
        pl.kernel,
        mesh=mesh,
        out_type=(
            jax.ShapeDtypeStruct((ep_pad, 16), jnp.float32),
            jax.ShapeDtypeStruct((NCORE, mpad, 128), jnp.float32),
        ),
        scratch_types=[
            pltpu.VMEM((KA,), jnp.int32),
            pltpu.VMEM((KA,), jnp.int32),
            pltpu.VMEM((KA, 128), jnp.float32),
            pltpu.VMEM((KA, 128), jnp.float32),
            pltpu.VMEM((KA, 16), jnp.float32),
            pltpu.VMEM((KA, 128), jnp.float32),
            pltpu.VMEM((1, 128), jnp.float32),
            pltpu.VMEM((1, 128), jnp.float32),
            pltpu.VMEM_SHARED((nacc, 128), jnp.float32),
            pltpu.SemaphoreType.DMA,
            pltpu.SemaphoreType.DMA,
        ],
    )
    def sca(src_hbm, dst_hbm, auxs_hbm, auxd_hbm, maxs_hbm, maxd_hbm,
            z128_hbm, coeff_hbm, denp_hbm,
            idxs_v, idxd_v, as_v, ad_v, co_v, co128_v, ms_v, md_v, den_sh,
            sem_a, sem_b):
        c = lax.axis_index("c")
        s = lax.axis_index("s")
        wid = c * NSUB + s

        pltpu.sync_copy(maxs_hbm, ms_v)
        pltpu.sync_copy(maxd_hbm, md_v)
        lanes = lax.iota(jnp.int32, 16)
        cvec = jnp.maximum(ms_v[0, 0:16] + md_v[0, 0:16], 0.0)
        cvec = jnp.where(lanes < heads, cvec, jnp.float32(1e30))

        def zrow(e, _):
            for j in range(8):
                co128_v[e, j * 16:(j + 1) * 16] = jnp.zeros((16,), jnp.float32)
            return 0
        lax.fori_loop(0, KA, zrow, 0)
        zoff = 0
        for zr in zslabs:
            pltpu.sync_copy(z128_hbm.at[pl.ds(0, zr)],
                            den_sh.at[pl.ds(s * rows_per_sub + zoff, zr)])
            zoff += zr
        plsc.subcore_barrier()

        def chunk(i, _):
            base = (wid * cpw + i) * KA
            pltpu.sync_copy(src_hbm.at[pl.ds(base, KA)], idxs_v)
            pltpu.sync_copy(dst_hbm.at[pl.ds(base, KA)], idxd_v)
            ga = pltpu.async_copy(auxs_hbm.at[idxs_v], as_v, sem_a)
            gb = pltpu.async_copy(auxd_hbm.at[idxd_v], ad_v, sem_b)
            ga.wait()
            gb.wait()

            def edge(e, _):
                al = as_v[e, 0:16] + ad_v[e, 0:16]
                al = jnp.where(al >= 0.0, al, 0.2 * al)
                cf = jnp.exp(al - cvec)
                valid = (base + e < ep).astype(jnp.float32)
                cf = cf * valid
                co_v[e, :] = cf
                co128_v[e, 0:16] = cf
                return 0

            lax.fori_loop(0, KA, edge, 0)
            pltpu.sync_copy(co128_v, den_sh.at[idxd_v], add=True)
            pltpu.sync_copy(co_v, coeff_hbm.at[pl.ds(base, KA)])
            return 0

        lax.fori_loop(0, cpw, chunk, 0)
        plsc.subcore_barrier()
        for cval in range(NCORE):
            @pl.when(c == cval)
            def _(cval=cval):
                pltpu.sync_copy(
                    den_sh.at[pl.ds(s * rows_per_sub, rows_per_sub)],
                    denp_hbm.at[cval].at[pl.ds(s * rows_per_sub,
                                               rows_per_sub)])

    return sca


def _make_scb1(ep_pad, mpad, nacc):
    cpw = ep_pad // (K * NSUB)
    rows_per_sub = nacc // NSUB
    zslabs = [K] * (rows_per_sub // K) + ([rows_per_sub % K] if rows_per_sub % K else [])
    mesh = plsc.VectorSubcoreMesh(core_axis_name="c", subcore_axis_name="s")

    @functools.partial(
        pl.kernel,
        mesh=mesh,
        out_type=jax.ShapeDtypeStruct((4, mpad, 128), jnp.float32),
        scratch_types=[
            pltpu.VMEM((K,), jnp.int32),
            pltpu.VMEM((K,), jnp.int32),
            pltpu.VMEM((K, 16), jnp.float32),
            pltpu.VMEM((K, 128), jnp.float32),
            pltpu.VMEM((K, 128), jnp.float32),
            pltpu.VMEM_SHARED((nacc, 128), jnp.float32),
            pltpu.SemaphoreType.DMA,
        ],
    )
    def scb1(h0_hbm, h1_hbm, h2_hbm, h3_hbm, coeff_hbm, src_hbm, dst_hbm,
             z128_hbm, num_hbm,
             idxs_v, idxd_v, co_v, rows_v, scaled_v, acc_sh, semg):
        c = lax.axis_index("c")
        s = lax.axis_index("s")
        tables = (h0_hbm, h1_hbm, h2_hbm, h3_hbm)

        for p in range(2):
            zoff = 0
            for zr in zslabs:
                pltpu.sync_copy(
                    z128_hbm.at[pl.ds(0, zr)],
                    acc_sh.at[pl.ds(s * rows_per_sub + zoff, zr)])
                zoff += zr
            plsc.subcore_barrier()

            for cval in range(NCORE):
                @pl.when(c == cval)
                def _(cval=cval, p=p):
                    head = 2 * cval + p
                    table = tables[head]

                    def chunk(i, _):
                        base = (s * cpw + i) * K
                        pltpu.sync_copy(src_hbm.at[pl.ds(base, K)], idxs_v)
                        pltpu.sync_copy(dst_hbm.at[pl.ds(base, K)], idxd_v)
                        pltpu.sync_copy(coeff_hbm.at[pl.ds(base, K)], co_v)
                        pltpu.async_copy(table.at[idxs_v], rows_v,
                                         semg).wait()

                        def edge(e, _):
                            cf = jnp.full((16,), co_v[e, :][head], jnp.float32)
                            for j in range(8):
                                scaled_v[e, j * 16:(j + 1) * 16] = (
                                    cf * rows_v[e, j * 16:(j + 1) * 16])
                            return 0

                        lax.fori_loop(0, K, edge, 0)
                        pltpu.sync_copy(scaled_v, acc_sh.at[idxd_v],
                                        add=True)
                        return 0

                    lax.fori_loop(0, cpw, chunk, 0)

            plsc.subcore_barrier()
            for cval in range(NCORE):
                @pl.when(c == cval)
                def _(cval=cval, p=p):
                    pltpu.sync_copy(
                        acc_sh.at[pl.ds(s * rows_per_sub, rows_per_sub)],
                        num_hbm.at[2 * cval + p].at[
                            pl.ds(s * rows_per_sub, rows_per_sub)])
            plsc.subcore_barrier()

    return scb1


def _make_scb2(ep_pad, mpad, nacc):
    cpw = ep_pad // (K * NCORE * NSUB)
    rows_per_sub = nacc // NSUB
    zslabs = [K] * (rows_per_sub // K) + ([rows_per_sub % K] if rows_per_sub % K else [])
    mesh = plsc.VectorSubcoreMesh(core_axis_name="c", subcore_axis_name="s")

    @functools.partial(
        pl.kernel,
        mesh=mesh,
        out_type=jax.ShapeDtypeStruct((NCORE, mpad, 128), jnp.float32),
        scratch_types=[
            pltpu.VMEM((K,), jnp.int32),
            pltpu.VMEM((K,), jnp.int32),
            pltpu.VMEM((K, 16), jnp.float32),
            pltpu.VMEM((K, 128), jnp.float32),
            pltpu.VMEM((K, 128), jnp.float32),
            pltpu.VMEM_SHARED((nacc, 128), jnp.float32),
            pltpu.SemaphoreType.DMA,
        ],
    )
    def scb2(h_hbm, coeff_hbm, src_hbm, dst_hbm, z128_hbm, nump_hbm,
             idxs_v, idxd_v, co_v, rows_v, scaled_v, acc_sh, semg):
        c = lax.axis_index("c")
        s = lax.axis_index("s")
        wid = c * NSUB + s

        zoff = 0
        for zr in zslabs:
            pltpu.sync_copy(
                z128_hbm.at[pl.ds(0, zr)],
                acc_sh.at[pl.ds(s * rows_per_sub + zoff, zr)])
            zoff += zr
        plsc.subcore_barrier()

        def chunk(i, _):
            base = (wid * cpw + i) * K
            pltpu.sync_copy(src_hbm.at[pl.ds(base, K)], idxs_v)
            pltpu.sync_copy(dst_hbm.at[pl.ds(base, K)], idxd_v)
            pltpu.sync_copy(coeff_hbm.at[pl.ds(base, K)], co_v)
            pltpu.async_copy(h_hbm.at[idxs_v], rows_v, semg).wait()

            def edge(e, _):
                cf = jnp.full((16,), co_v[e, :][0], jnp.float32)
                for j in range(8):
                    scaled_v[e, j * 16:(j + 1) * 16] = (
                        cf * rows_v[e, j * 16:(j + 1) * 16])
                return 0

            lax.fori_loop(0, K, edge, 0)
            pltpu.sync_copy(scaled_v, acc_sh.at[idxd_v], add=True)
            return 0

        lax.fori_loop(0, cpw, chunk, 0)
        plsc.subcore_barrier()
        for cval in range(NCORE):
            @pl.when(c == cval)
            def _(cval=cval):
                pltpu.sync_copy(
                    acc_sh.at[pl.ds(s * rows_per_sub, rows_per_sub)],
                    nump_hbm.at[cval].at[pl.ds(s * rows_per_sub,
                                               rows_per_sub)])

    return scb2


def _tc2_body(nvalid, num_ref, den_ref, b1_ref, w_ref,
              h2m, auxs2, auxd2, maxs2, maxd2):
    den = den_ref[0] + den_ref[1]
    parts = []
    for hh in range(4):
        dh = den[:, hh:hh + 1] + 1e-16
        parts.append(num_ref[hh] / dh)
    h1b = jnp.concatenate(parts, axis=1) + b1_ref[...]
    h1b = jnp.maximum(h1b, 0.0)
    i = pl.program_id(0)
    rowid = lax.broadcasted_iota(jnp.int32, (BM, 1), 0) + i * BM
    h1b = jnp.where(rowid < nvalid, h1b, 0.0)
    w = w_ref[...]
    h2m[...] = jnp.dot(h1b, w[:, :128], preferred_element_type=jnp.float32)
    a_s = jnp.dot(h1b, w[:, 128:256], preferred_element_type=jnp.float32)
    a_d = jnp.dot(h1b, w[:, 256:384], preferred_element_type=jnp.float32)
    auxs2[...] = a_s
    auxd2[...] = a_d
    ms = jnp.max(a_s, axis=0, keepdims=True)
    md = jnp.max(a_d, axis=0, keepdims=True)

    @pl.when(i == 0)
    def _():
        maxs2[...] = ms
        maxd2[...] = md

    @pl.when(i > 0)
    def _():
        maxs2[...] = jnp.maximum(maxs2[...], ms)
        maxd2[...] = jnp.maximum(maxd2[...], md)


def _tc2(num, denp, b1_2d, wall2, nvalid, mpad):
    return pl.pallas_call(
        functools.partial(_tc2_body, nvalid),
        grid=(mpad // BM,),
        in_specs=[
            pl.BlockSpec((4, BM, 128), lambda i: (0, i, 0)),
            pl.BlockSpec((2, BM, 128), lambda i: (0, i, 0)),
            pl.BlockSpec((1, 512), lambda i: (0, 0)),
            pl.BlockSpec((512, 384), lambda i: (0, 0)),
        ],
        out_specs=[
            pl.BlockSpec((BM, 128), lambda i: (i, 0)),
            pl.BlockSpec((BM, 128), lambda i: (i, 0)),
            pl.BlockSpec((BM, 128), lambda i: (i, 0)),
            pl.BlockSpec((1, 128), lambda i: (0, 0)),
            pl.BlockSpec((1, 128), lambda i: (0, 0)),
        ],
        out_shape=[
            jax.ShapeDtypeStruct((mpad, 128), jnp.float32),
            jax.ShapeDtypeStruct((mpad, 128), jnp.float32),
            jax.ShapeDtypeStruct((mpad, 128), jnp.float32),
            jax.ShapeDtypeStruct((1, 128), jnp.float32),
            jax.ShapeDtypeStruct((1, 128), jnp.float32),
        ],
    )(num, denp, b1_2d, wall2)


def _tc3_body(nvalid, nsteps, nump_ref, denp_ref, b2_ref,
              fc1w_ref, fc1b_ref, fc2w_ref, fc2b_ref, out_ref, acc):
    den = denp_ref[0] + denp_ref[1]
    num = nump_ref[0] + nump_ref[1]
    h2 = jnp.maximum(num / (den[:, 0:1] + 1e-16) + b2_ref[...], 0.0)
    i = pl.program_id(0)
    rowid = lax.broadcasted_iota(jnp.int32, (BM, 1), 0) + i * BM
    h2 = jnp.where(rowid < nvalid, h2, 0.0)
    psum = jnp.sum(h2, axis=0, keepdims=True)

    @pl.when(i == 0)
    def _():
        acc[...] = psum

    @pl.when(i > 0)
    def _():
        acc[...] = acc[...] + psum

    @pl.when(i == nsteps - 1)
    def _():
        pooled = acc[...] / jnp.float32(nvalid)
        hid = jnp.maximum(
            jnp.dot(pooled, fc1w_ref[...],
                    preferred_element_type=jnp.float32) + fc1b_ref[...], 0.0)
        out_ref[...] = (
            jnp.dot(hid, fc2w_ref[...], preferred_element_type=jnp.float32)
            + fc2b_ref[...])


def _tc3(nump, denp, b2_2d, fc1w, fc1b_2d, fc2wp, fc2bp, nvalid, mpad):
    nsteps = mpad // BM
    return pl.pallas_call(
        functools.partial(_tc3_body, nvalid, nsteps),
        grid=(nsteps,),
        in_specs=[
            pl.BlockSpec((2, BM, 128), lambda i: (0, i, 0)),
            pl.BlockSpec((2, BM, 128), lambda i: (0, i, 0)),
            pl.BlockSpec((1, 128), lambda i: (0, 0)),
            pl.BlockSpec((128, 64), lambda i: (0, 0)),
            pl.BlockSpec((1, 64), lambda i: (0, 0)),
            pl.BlockSpec((64, 128), lambda i: (0, 0)),
            pl.BlockSpec((1, 128), lambda i: (0, 0)),
        ],
        out_specs=pl.BlockSpec((1, 128), lambda i: (0, 0)),
        out_shape=jax.ShapeDtypeStruct((1, 128), jnp.float32),
        scratch_shapes=[pltpu.VMEM((1, 128), jnp.float32)],
    )(nump, denp, b2_2d, fc1w, fc1b_2d, fc2wp, fc2bp)


def kernel(x, edge_index, W1, att_src1, att_dst1, b1, W2, att_src2,
           att_dst2, b2, fc1_w, fc1_b, fc2_w, fc2_b):
    n = x.shape[0]
    kdim = x.shape[1]
    e_raw = edge_index.shape[1]
    ep = e_raw + n
    mpad = ((n + BM - 1) // BM) * BM
    wchunk = K * NCORE * NSUB
    ep_pad = ((ep + wchunk - 1) // wchunk) * wchunk
    kp = ((kdim + 127) // 128) * 128

    loop = jnp.arange(n, dtype=jnp.int32)
    zpad = jnp.zeros((ep_pad - ep,), jnp.int32)
    src = jnp.concatenate([edge_index[0].astype(jnp.int32), loop, zpad])
    dst = jnp.concatenate([edge_index[1].astype(jnp.int32), loop, zpad])

    xpad = jnp.pad(x, ((0, mpad - n), (0, kp - kdim)))
    w1r = W1.reshape(kdim, 4, 128)
    wauxs1 = jnp.einsum("khc,hc->kh", w1r, att_src1[0])
    wauxd1 = jnp.einsum("khc,hc->kh", w1r, att_dst1[0])
    wall = jnp.zeros((kp, 768), jnp.float32)
    wall = wall.at[:kdim, :512].set(W1)
    wall = wall.at[:kdim, 512:516].set(wauxs1)
    wall = wall.at[:kdim, 640:644].set(wauxd1)

    wauxs2 = jnp.einsum("kc,c->k", W2, att_src2[0, 0])
    wauxd2 = jnp.einsum("kc,c->k", W2, att_dst2[0, 0])
    wall2 = jnp.zeros((512, 384), jnp.float32)
    wall2 = wall2.at[:, :128].set(W2)
    wall2 = wall2.at[:, 128].set(wauxs2)
    wall2 = wall2.at[:, 256].set(wauxd2)

    z128 = jnp.zeros((K, 128), jnp.float32)

    h0, h1_, h2_, h3, auxs, auxd, maxs, maxd = _tc1(xpad, wall, mpad)
    nacc = ((n + 127) // 128) * 128
    sca = _make_sca(ep, ep_pad, mpad, nacc, heads=4)
    coeff, denp = sca(src, dst, auxs, auxd, maxs, maxd, z128)
    scb1 = _make_scb1(ep_pad, mpad, nacc)
    num = scb1(h0, h1_, h2_, h3, coeff, src, dst, z128)

    b1_2d = b1.reshape(1, 512)
    h2m, auxs2, auxd2, maxs2, maxd2 = _tc2(num, denp, b1_2d, wall2, n, mpad)
    sca2 = _make_sca(ep, ep_pad, mpad, nacc, heads=1)
    coeff2, denp2 = sca2(src, dst, auxs2, auxd2, maxs2, maxd2, z128)
    scb2 = _make_scb2(ep_pad, mpad, nacc)
    nump2 = scb2(h2m, coeff2, src, dst, z128)

    b2_2d = b2.reshape(1, 128)
    fc1b_2d = fc1_b.reshape(1, 64)
    fc2wp = jnp.pad(fc2_w, ((0, 0), (0, 128 - fc2_w.shape[1])))
    fc2bp = jnp.pad(fc2_b, (0, 128 - fc2_b.shape[0])).reshape(1, 128)
    out = _tc3(nump2, denp2, b2_2d, fc1_w, fc1b_2d, fc2wp, fc2bp, n, mpad)
    return out[:, :fc2_w.shape[1]]


# speedup vs baseline: 1.2140x; 1.2140x over previous
import functools

import jax
import jax.numpy as jnp
from jax import lax
from jax.experimental import pallas as pl
from jax.experimental.pallas import tpu as pltpu
from jax.experimental.pallas import tpu_sc as plsc

BM = 256
K = 128
KA = 64
NSUB = 16
NCORE = 2


def _tc1_body(x_ref, w_ref, h0, h1, h2, h3, auxs, auxd, maxs, maxd):
    xb = x_ref[...]
    w = w_ref[...]
    for hh, href in enumerate((h0, h1, h2, h3)):
        href[...] = jnp.dot(xb, w[:, hh * 128:(hh + 1) * 128],
                            preferred_element_type=jnp.float32)
    a_s = jnp.dot(xb, w[:, 512:640], preferred_element_type=jnp.float32)
    a_d = jnp.dot(xb, w[:, 640:768], preferred_element_type=jnp.float32)
    auxs[...] = a_s
    auxd[...] = a_d
    ms = jnp.max(a_s, axis=0, keepdims=True)
    md = jnp.max(a_d, axis=0, keepdims=True)
    i = pl.program_id(0)

    @pl.when(i == 0)
    def _():
        maxs[...] = ms
        maxd[...] = md

    @pl.when(i > 0)
    def _():
        maxs[...] = jnp.maximum(maxs[...], ms)
        maxd[...] = jnp.maximum(maxd[...], md)


def _tc1(xpad, wall, mpad):
    kp = xpad.shape[1]
    return pl.pallas_call(
        _tc1_body,
        grid=(mpad // BM,),
        in_specs=[
            pl.BlockSpec((BM, kp), lambda i: (i, 0)),
            pl.BlockSpec((kp, 768), lambda i: (0, 0)),
        ],
        out_specs=[
            pl.BlockSpec((BM, 128), lambda i: (i, 0)),
            pl.BlockSpec((BM, 128), lambda i: (i, 0)),
            pl.BlockSpec((BM, 128), lambda i: (i, 0)),
            pl.BlockSpec((BM, 128), lambda i: (i, 0)),
            pl.BlockSpec((BM, 128), lambda i: (i, 0)),
            pl.BlockSpec((BM, 128), lambda i: (i, 0)),
            pl.BlockSpec((1, 128), lambda i: (0, 0)),
            pl.BlockSpec((1, 128), lambda i: (0, 0)),
        ],
        out_shape=[
            jax.ShapeDtypeStruct((mpad, 128), jnp.float32),
            jax.ShapeDtypeStruct((mpad, 128), jnp.float32),
            jax.ShapeDtypeStruct((mpad, 128), jnp.float32),
            jax.ShapeDtypeStruct((mpad, 128), jnp.float32),
            jax.ShapeDtypeStruct((mpad, 128), jnp.float32),
            jax.ShapeDtypeStruct((mpad, 128), jnp.float32),
            jax.ShapeDtypeStruct((1, 128), jnp.float32),
            jax.ShapeDtypeStruct((1, 128), jnp.float32),
        ],
    )(xpad, wall)


def _make_sca(ep, ep_pad, mpad, nacc, heads):
    cpw = ep_pad // (KA * NCORE * NSUB)
    rows_per_sub = nacc // NSUB
    zslabs = [K] * (rows_per_sub // K) + ([rows_per_sub % K] if rows_per_sub % K else [])
    assert cpw % 2 == 0
    mesh = plsc.VectorSubcoreMesh(core_axis_name="c", subcore_axis_name="s")

    @functools.partial(
        pl.kernel,
        mesh=mesh,
        out_type=(
            jax.ShapeDtypeStruct((ep_pad, 16), jnp.float32),
            jax.ShapeDtypeStruct((NCORE, mpad, 128), jnp.float32),
        ),
        scratch_types=[
            pltpu.VMEM((KA,), jnp.int32),
            pltpu.VMEM((KA,), jnp.int32),
            pltpu.VMEM((KA,), jnp.int32),
            pltpu.VMEM((KA,), jnp.int32),
            pltpu.VMEM((KA, 128), jnp.float32),
            pltpu.VMEM((KA, 128), jnp.float32),
            pltpu.VMEM((KA, 128), jnp.float32),
            pltpu.VMEM((KA, 128), jnp.float32),
            pltpu.VMEM((KA, 16), jnp.float32),
            pltpu.VMEM((KA, 128), jnp.float32),
            pltpu.VMEM((1, 128), jnp.float32),
            pltpu.VMEM((1, 128), jnp.float32),
            pltpu.VMEM_SHARED((nacc, 128), jnp.float32),
            pltpu.SemaphoreType.DMA,
            pltpu.SemaphoreType.DMA,
            pltpu.SemaphoreType.DMA,
            pltpu.SemaphoreType.DMA,
        ],
    )
    def sca(src_hbm, dst_hbm, auxs_hbm, auxd_hbm, maxs_hbm, maxd_hbm,
            z128_hbm, coeff_hbm, denp_hbm,
            idxs0_v, idxs1_v, idxd0_v, idxd1_v, as0_v, as1_v, ad0_v, ad1_v,
            co_v, co128_v, ms_v, md_v, den_sh,
            sa0, sa1, sb0, sb1):
        c = lax.axis_index("c")
        s = lax.axis_index("s")
        wid = c * NSUB + s
        idxs_b = (idxs0_v, idxs1_v)
        idxd_b = (idxd0_v, idxd1_v)
        as_b = (as0_v, as1_v)
        ad_b = (ad0_v, ad1_v)
        sa = (sa0, sa1)
        sb = (sb0, sb1)

        pltpu.sync_copy(maxs_hbm, ms_v)
        pltpu.sync_copy(maxd_hbm, md_v)
        lanes = lax.iota(jnp.int32, 16)
        cvec = jnp.maximum(ms_v[0, 0:16] + md_v[0, 0:16], 0.0)
        cvec = jnp.where(lanes < heads, cvec, jnp.float32(1e30))

        def zrow(e, _):
            for j in range(8):
                co128_v[e, j * 16:(j + 1) * 16] = jnp.zeros((16,), jnp.float32)
            return 0
        lax.fori_loop(0, KA, zrow, 0)
        zoff = 0
        for zr in zslabs:
            pltpu.sync_copy(z128_hbm.at[pl.ds(0, zr)],
                            den_sh.at[pl.ds(s * rows_per_sub + zoff, zr)])
            zoff += zr
        plsc.subcore_barrier()

        cbase = wid * cpw

        def start(i, buf):
            base = (cbase + i) * KA
            pltpu.sync_copy(src_hbm.at[pl.ds(base, KA)], idxs_b[buf])
            pltpu.sync_copy(dst_hbm.at[pl.ds(base, KA)], idxd_b[buf])
            pltpu.async_copy(auxs_hbm.at[idxs_b[buf]], as_b[buf], sa[buf])
            pltpu.async_copy(auxd_hbm.at[idxd_b[buf]], ad_b[buf], sb[buf])

        start(0, 0)

        def pair(g, _):
            for bb in range(2):
                i = 2 * g + bb
                nb = 1 - bb

                @pl.when(i + 1 < cpw)
                def _():
                    start(i + 1, nb)

                pltpu.make_async_copy(
                    auxs_hbm.at[idxs_b[bb]], as_b[bb], sa[bb]).wait()
                pltpu.make_async_copy(
                    auxd_hbm.at[idxd_b[bb]], ad_b[bb], sb[bb]).wait()
                base = (cbase + i) * KA

                def edge(e, _):
                    al = as_b[bb][e, 0:16] + ad_b[bb][e, 0:16]
                    al = jnp.where(al >= 0.0, al, 0.2 * al)
                    cf = jnp.exp(al - cvec)
                    valid = (base + e < ep).astype(jnp.float32)
                    cf = cf * valid
                    co_v[e, :] = cf
                    co128_v[e, 0:16] = cf
                    return 0

                lax.fori_loop(0, KA, edge, 0)
                pltpu.sync_copy(co128_v, den_sh.at[idxd_b[bb]], add=True)
                pltpu.sync_copy(co_v, coeff_hbm.at[pl.ds(base, KA)])
            return 0

        lax.fori_loop(0, cpw // 2, pair, 0)
        plsc.subcore_barrier()
        for cval in range(NCORE):
            @pl.when(c == cval)
            def _(cval=cval):
                pltpu.sync_copy(
                    den_sh.at[pl.ds(s * rows_per_sub, rows_per_sub)],
                    denp_hbm.at[cval].at[pl.ds(s * rows_per_sub,
                                               rows_per_sub)])

    return sca


def _scb_scratch(nacc):
    return [
        pltpu.VMEM((K,), jnp.int32),
        pltpu.VMEM((K,), jnp.int32),
        pltpu.VMEM((K,), jnp.int32),
        pltpu.VMEM((K,), jnp.int32),
        pltpu.VMEM((K, 16), jnp.float32),
        pltpu.VMEM((K, 128), jnp.float32),
        pltpu.VMEM((K, 128), jnp.float32),
        pltpu.VMEM_SHARED((nacc, 128), jnp.float32),
        pltpu.SemaphoreType.DMA,
        pltpu.SemaphoreType.DMA,
    ]


def _scb_pipeline(table, coeff_hbm, src_hbm, dst_hbm, acc_sh, head,
                  cbase, cpw, idxs_b, idxd_b, rows_b, co_v, sg):
    def start(i, buf):
        base = (cbase + i) * K
        pltpu.sync_copy(src_hbm.at[pl.ds(base, K)], idxs_b[buf])
        pltpu.sync_copy(dst_hbm.at[pl.ds(base, K)], idxd_b[buf])
        pltpu.async_copy(table.at[idxs_b[buf]], rows_b[buf], sg[buf])

    start(0, 0)

    def pair(g, _):
        for bb in range(2):
            i = 2 * g + bb
            nb = 1 - bb

            @pl.when(i + 1 < cpw)
            def _():
                start(i + 1, nb)

            pltpu.make_async_copy(
                table.at[idxs_b[bb]], rows_b[bb], sg[bb]).wait()
            base = (cbase + i) * K
            pltpu.sync_copy(coeff_hbm.at[pl.ds(base, K)], co_v)

            def edge(e, _):
                cf = jnp.full((16,), co_v[e, :][head], jnp.float32)
                for j in range(8):
                    rows_b[bb][e, j * 16:(j + 1) * 16] = (
                        cf * rows_b[bb][e, j * 16:(j + 1) * 16])
                return 0

            lax.fori_loop(0, K, edge, 0)
            pltpu.sync_copy(rows_b[bb], acc_sh.at[idxd_b[bb]], add=True)
        return 0

    lax.fori_loop(0, cpw // 2, pair, 0)


def _make_scb1(ep_pad, mpad, nacc):
    cpw = ep_pad // (K * NSUB)
    rows_per_sub = nacc // NSUB
    zslabs = [K] * (rows_per_sub // K) + ([rows_per_sub % K] if rows_per_sub % K else [])
    assert cpw % 2 == 0
    mesh = plsc.VectorSubcoreMesh(core_axis_name="c", subcore_axis_name="s")

    @functools.partial(
        pl.kernel,
        mesh=mesh,
        out_type=jax.ShapeDtypeStruct((4, mpad, 128), jnp.float32),
        scratch_types=_scb_scratch(nacc),
    )
    def scb1(h0_hbm, h1_hbm, h2_hbm, h3_hbm, coeff_hbm, src_hbm, dst_hbm,
             z128_hbm, num_hbm,
             idxs0_v, idxs1_v, idxd0_v, idxd1_v, co_v, rows0_v, rows1_v,
             acc_sh, sg0, sg1):
        c = lax.axis_index("c")
        s = lax.axis_index("s")
        tables = (h0_hbm, h1_hbm, h2_hbm, h3_hbm)
        idxs_b = (idxs0_v, idxs1_v)
        idxd_b = (idxd0_v, idxd1_v)
        rows_b = (rows0_v, rows1_v)
        sg = (sg0, sg1)

        for p in range(2):
            zoff = 0
            for zr in zslabs:
                pltpu.sync_copy(
                    z128_hbm.at[pl.ds(0, zr)],
                    acc_sh.at[pl.ds(s * rows_per_sub + zoff, zr)])
                zoff += zr
            plsc.subcore_barrier()

            for cval in range(NCORE):
                @pl.when(c == cval)
                def _(cval=cval, p=p):
                    head = 2 * cval + p
                    _scb_pipeline(tables[head], coeff_hbm, src_hbm, dst_hbm,
                                  acc_sh, head, s * cpw, cpw,
                                  idxs_b, idxd_b, rows_b, co_v, sg)

            plsc.subcore_barrier()
            for cval in range(NCORE):
                @pl.when(c == cval)
                def _(cval=cval, p=p):
                    pltpu.sync_copy(
                        acc_sh.at[pl.ds(s * rows_per_sub, rows_per_sub)],
                        num_hbm.at[2 * cval + p].at[
                            pl.ds(s * rows_per_sub, rows_per_sub)])
            plsc.subcore_barrier()

    return scb1


def _make_scb2(ep_pad, mpad, nacc):
    cpw = ep_pad // (K * NCORE * NSUB)
    rows_per_sub = nacc // NSUB
    zslabs = [K] * (rows_per_sub // K) + ([rows_per_sub % K] if rows_per_sub % K else [])
    assert cpw % 2 == 0
    mesh = plsc.VectorSubcoreMesh(core_axis_name="c", subcore_axis_name="s")

    @functools.partial(
        pl.kernel,
        mesh=mesh,
        out_type=jax.ShapeDtypeStruct((NCORE, mpad, 128), jnp.float32),
        scratch_types=_scb_scratch(nacc),
    )
    def scb2(h_hbm, coeff_hbm, src_hbm, dst_hbm, z128_hbm, nump_hbm,
             idxs0_v, idxs1_v, idxd0_v, idxd1_v, co_v, rows0_v, rows1_v,
             acc_sh, sg0, sg1):
        c = lax.axis_index("c")
        s = lax.axis_index("s")
        wid = c * NSUB + s
        idxs_b = (idxs0_v, idxs1_v)
        idxd_b = (idxd0_v, idxd1_v)
        rows_b = (rows0_v, rows1_v)
        sg = (sg0, sg1)

        zoff = 0
        for zr in zslabs:
            pltpu.sync_copy(
                z128_hbm.at[pl.ds(0, zr)],
                acc_sh.at[pl.ds(s * rows_per_sub + zoff, zr)])
            zoff += zr
        plsc.subcore_barrier()

        _scb_pipeline(h_hbm, coeff_hbm, src_hbm, dst_hbm, acc_sh, 0,
                      wid * cpw, cpw, idxs_b, idxd_b, rows_b, co_v, sg)

        plsc.subcore_barrier()
        for cval in range(NCORE):
            @pl.when(c == cval)
            def _(cval=cval):
                pltpu.sync_copy(
                    acc_sh.at[pl.ds(s * rows_per_sub, rows_per_sub)],
                    nump_hbm.at[cval].at[pl.ds(s * rows_per_sub,
                                               rows_per_sub)])

    return scb2


def _tc2_body(nvalid, num_ref, den_ref, b1_ref, w_ref,
              h2m, auxs2, auxd2, maxs2, maxd2):
    den = den_ref[0] + den_ref[1]
    parts = []
    for hh in range(4):
        dh = den[:, hh:hh + 1] + 1e-16
        parts.append(num_ref[hh] / dh)
    h1b = jnp.concatenate(parts, axis=1) + b1_ref[...]
    h1b = jnp.maximum(h1b, 0.0)
    i = pl.program_id(0)
    rowid = lax.broadcasted_iota(jnp.int32, (BM, 1), 0) + i * BM
    h1b = jnp.where(rowid < nvalid, h1b, 0.0)
    w = w_ref[...]
    h2m[...] = jnp.dot(h1b, w[:, :128], preferred_element_type=jnp.float32)
    a_s = jnp.dot(h1b, w[:, 128:256], preferred_element_type=jnp.float32)
    a_d = jnp.dot(h1b, w[:, 256:384], preferred_element_type=jnp.float32)
    auxs2[...] = a_s
    auxd2[...] = a_d
    ms = jnp.max(a_s, axis=0, keepdims=True)
    md = jnp.max(a_d, axis=0, keepdims=True)

    @pl.when(i == 0)
    def _():
        maxs2[...] = ms
        maxd2[...] = md

    @pl.when(i > 0)
    def _():
        maxs2[...] = jnp.maximum(maxs2[...], ms)
        maxd2[...] = jnp.maximum(maxd2[...], md)


def _tc2(num, denp, b1_2d, wall2, nvalid, mpad):
    return pl.pallas_call(
        functools.partial(_tc2_body, nvalid),
        grid=(mpad // BM,),
        in_specs=[
            pl.BlockSpec((4, BM, 128), lambda i: (0, i, 0)),
            pl.BlockSpec((2, BM, 128), lambda i: (0, i, 0)),
            pl.BlockSpec((1, 512), lambda i: (0, 0)),
            pl.BlockSpec((512, 384), lambda i: (0, 0)),
        ],
        out_specs=[
            pl.BlockSpec((BM, 128), lambda i: (i, 0)),
            pl.BlockSpec((BM, 128), lambda i: (i, 0)),
            pl.BlockSpec((BM, 128), lambda i: (i, 0)),
            pl.BlockSpec((1, 128), lambda i: (0, 0)),
            pl.BlockSpec((1, 128), lambda i: (0, 0)),
        ],
        out_shape=[
            jax.ShapeDtypeStruct((mpad, 128), jnp.float32),
            jax.ShapeDtypeStruct((mpad, 128), jnp.float32),
            jax.ShapeDtypeStruct((mpad, 128), jnp.float32),
            jax.ShapeDtypeStruct((1, 128), jnp.float32),
            jax.ShapeDtypeStruct((1, 128), jnp.float32),
        ],
    )(num, denp, b1_2d, wall2)


def _tc3_body(nvalid, nsteps, nump_ref, denp_ref, b2_ref,
              fc1w_ref, fc1b_ref, fc2w_ref, fc2b_ref, out_ref, acc):
    den = denp_ref[0] + denp_ref[1]
    num = nump_ref[0] + nump_ref[1]
    h2 = jnp.maximum(num / (den[:, 0:1] + 1e-16) + b2_ref[...], 0.0)
    i = pl.program_id(0)
    rowid = lax.broadcasted_iota(jnp.int32, (BM, 1), 0) + i * BM
    h2 = jnp.where(rowid < nvalid, h2, 0.0)
    psum = jnp.sum(h2, axis=0, keepdims=True)

    @pl.when(i == 0)
    def _():
        acc[...] = psum

    @pl.when(i > 0)
    def _():
        acc[...] = acc[...] + psum

    @pl.when(i == nsteps - 1)
    def _():
        pooled = acc[...] / jnp.float32(nvalid)
        hid = jnp.maximum(
            jnp.dot(pooled, fc1w_ref[...],
                    preferred_element_type=jnp.float32) + fc1b_ref[...], 0.0)
        out_ref[...] = (
            jnp.dot(hid, fc2w_ref[...], preferred_element_type=jnp.float32)
            + fc2b_ref[...])


def _tc3(nump, denp, b2_2d, fc1w, fc1b_2d, fc2wp, fc2bp, nvalid, mpad):
    nsteps = mpad // BM
    return pl.pallas_call(
        functools.partial(_tc3_body, nvalid, nsteps),
        grid=(nsteps,),
        in_specs=[
            pl.BlockSpec((2, BM, 128), lambda i: (0, i, 0)),
            pl.BlockSpec((2, BM, 128), lambda i: (0, i, 0)),
            pl.BlockSpec((1, 128), lambda i: (0, 0)),
            pl.BlockSpec((128, 64), lambda i: (0, 0)),
            pl.BlockSpec((1, 64), lambda i: (0, 0)),
            pl.BlockSpec((64, 128), lambda i: (0, 0)),
            pl.BlockSpec((1, 128), lambda i: (0, 0)),
        ],
        out_specs=pl.BlockSpec((1, 128), lambda i: (0, 0)),
        out_shape=jax.ShapeDtypeStruct((1, 128), jnp.float32),
        scratch_shapes=[pltpu.VMEM((1, 128), jnp.float32)],
    )(nump, denp, b2_2d, fc1w, fc1b_2d, fc2wp, fc2bp)


def kernel(x, edge_index, W1, att_src1, att_dst1, b1, W2, att_src2,
           att_dst2, b2, fc1_w, fc1_b, fc2_w, fc2_b):
    n = x.shape[0]
    kdim = x.shape[1]
    e_raw = edge_index.shape[1]
    ep = e_raw + n
    mpad = ((n + BM - 1) // BM) * BM
    wchunk = K * NCORE * NSUB
    ep_pad = ((ep + wchunk - 1) // wchunk) * wchunk
    kp = ((kdim + 127) // 128) * 128

    loop = jnp.arange(n, dtype=jnp.int32)
    zpad = jnp.zeros((ep_pad - ep,), jnp.int32)
    src = jnp.concatenate([edge_index[0].astype(jnp.int32), loop, zpad])
    dst = jnp.concatenate([edge_index[1].astype(jnp.int32), loop, zpad])

    xpad = jnp.pad(x, ((0, mpad - n), (0, kp - kdim)))
    w1r = W1.reshape(kdim, 4, 128)
    wauxs1 = jnp.einsum("khc,hc->kh", w1r, att_src1[0])
    wauxd1 = jnp.einsum("khc,hc->kh", w1r, att_dst1[0])
    wall = jnp.zeros((kp, 768), jnp.float32)
    wall = wall.at[:kdim, :512].set(W1)
    wall = wall.at[:kdim, 512:516].set(wauxs1)
    wall = wall.at[:kdim, 640:644].set(wauxd1)

    wauxs2 = jnp.einsum("kc,c->k", W2, att_src2[0, 0])
    wauxd2 = jnp.einsum("kc,c->k", W2, att_dst2[0, 0])
    wall2 = jnp.zeros((512, 384), jnp.float32)
    wall2 = wall2.at[:, :128].set(W2)
    wall2 = wall2.at[:, 128].set(wauxs2)
    wall2 = wall2.at[:, 256].set(wauxd2)

    z128 = jnp.zeros((K, 128), jnp.float32)

    h0, h1_, h2_, h3, auxs, auxd, maxs, maxd = _tc1(xpad, wall, mpad)
    nacc = ((n + 127) // 128) * 128
    sca = _make_sca(ep, ep_pad, mpad, nacc, heads=4)
    coeff, denp = sca(src, dst, auxs, auxd, maxs, maxd, z128)
    scb1 = _make_scb1(ep_pad, mpad, nacc)
    num = scb1(h0, h1_, h2_, h3, coeff, src, dst, z128)

    b1_2d = b1.reshape(1, 512)
    h2m, auxs2, auxd2, maxs2, maxd2 = _tc2(num, denp, b1_2d, wall2, n, mpad)
    sca2 = _make_sca(ep, ep_pad, mpad, nacc, heads=1)
    coeff2, denp2 = sca2(src, dst, auxs2, auxd2, maxs2, maxd2, z128)
    scb2 = _make_scb2(ep_pad, mpad, nacc)
    nump2 = scb2(h2m, coeff2, src, dst, z128)

    b2_2d = b2.reshape(1, 128)
    fc1b_2d = fc1_b.reshape(1, 64)
    fc2wp = jnp.pad(fc2_w, ((0, 0), (0, 128 - fc2_w.shape[1])))
    fc2bp = jnp.pad(fc2_b, (0, 128 - fc2_b.shape[0])).reshape(1, 128)
    out = _tc3(nump2, denp2, b2_2d, fc1_w, fc1b_2d, fc2wp, fc2bp, n, mpad)
    return out[:, :fc2_w.shape[1]]
